# baseline (speedup 1.0000x reference)
import jax
import jax.numpy as jnp
from jax import lax
from jax.experimental import pallas as pl
from jax.experimental.pallas import tpu as pltpu
from jax.experimental.pallas import tpu_sc as plsc

VOCAB = 1000000
EMBED_DIM = 64
BATCH = 16384
SEQ = 50

B = BATCH * SEQ
D = EMBED_DIM
NC, NS = 2, 16
NW = NC * NS
IDX_W = 128
CHUNK = 640
ROWS_PER_W = B // NW
STEPS = ROWS_PER_W // CHUNK
IDX_ROWS_PER_STEP = CHUNK // IDX_W
IDX_ROWS_PER_W = ROWS_PER_W // IDX_W


HALF = CHUNK // 2


def _gather_body(idx_hbm, table_hbm, out_hbm, idx_all, rows_v,
                 sem_g0a, sem_g0b, sem_g1a, sem_g1b, sem_o0, sem_o1):
    wid = lax.axis_index("s") * NC + lax.axis_index("c")
    row_base = wid * ROWS_PER_W
    sem_g = ((sem_g0a, sem_g0b), (sem_g1a, sem_g1b))
    sem_o = (sem_o0, sem_o1)

    idx0 = pl.multiple_of(wid * ROWS_PER_W, 8)
    pltpu.sync_copy(idx_hbm.at[pl.ds(idx0, ROWS_PER_W)], idx_all)

    def fire_gathers(g, buf):
        pltpu.async_copy(
            table_hbm.at[idx_all.at[pl.ds(g * CHUNK, HALF)]],
            rows_v.at[buf, pl.ds(0, HALF)],
            sem_g[buf][0],
        )
        pltpu.async_copy(
            table_hbm.at[idx_all.at[pl.ds(g * CHUNK + HALF, HALF)]],
            rows_v.at[buf, pl.ds(HALF, HALF)],
            sem_g[buf][1],
        )

    def wait_gathers(buf):
        pltpu.make_async_copy(
            table_hbm.at[pl.ds(0, HALF)], rows_v.at[buf, pl.ds(0, HALF)],
            sem_g[buf][0]
        ).wait()
        pltpu.make_async_copy(
            table_hbm.at[pl.ds(0, HALF)], rows_v.at[buf, pl.ds(HALF, HALF)],
            sem_g[buf][1]
        ).wait()

    def fire_writeback(g, buf):
        out0 = pl.multiple_of(row_base + g * CHUNK, CHUNK)
        pltpu.async_copy(rows_v.at[buf], out_hbm.at[pl.ds(out0, CHUNK)],
                         sem_o[buf])

    def drain_writeback(buf):
        pltpu.make_async_copy(
            rows_v.at[buf], out_hbm.at[pl.ds(0, CHUNK)], sem_o[buf]
        ).wait()

    fire_gathers(0, 0)

    def pair(i, carry):
        g0 = 2 * i
        g1 = 2 * i + 1

        @pl.when(i >= 1)
        def _():
            drain_writeback(1)

        fire_gathers(g1, 1)
        wait_gathers(0)
        fire_writeback(g0, 0)

        @pl.when(g1 + 1 < STEPS)
        def _():
            drain_writeback(0)
            fire_gathers(g1 + 1, 0)

        wait_gathers(1)
        fire_writeback(g1, 1)
        return carry

    lax.fori_loop(0, STEPS // 2, pair, 0)
    drain_writeback(0)
    drain_writeback(1)


@jax.jit
def _run(idx2d, table):
    mesh = plsc.VectorSubcoreMesh(core_axis_name="c", subcore_axis_name="s")
    k = pl.kernel(
        _gather_body,
        out_type=jax.ShapeDtypeStruct((B, D), jnp.float32),
        mesh=mesh,
        scratch_types=[
            pltpu.VMEM((ROWS_PER_W,), jnp.int32),
            pltpu.VMEM((2, CHUNK, D), jnp.float32),
            pltpu.SemaphoreType.DMA,
            pltpu.SemaphoreType.DMA,
            pltpu.SemaphoreType.DMA,
            pltpu.SemaphoreType.DMA,
            pltpu.SemaphoreType.DMA,
            pltpu.SemaphoreType.DMA,
        ],
        compiler_params=pltpu.CompilerParams(use_tc_tiling_on_sc=False),
    )
    return k(idx2d, table)


def kernel(chord_pitches, pitch_table):
    idx1d = chord_pitches.reshape(B)
    out = _run(idx1d, pitch_table)
    return out.reshape(BATCH, SEQ, D)


# speedup vs baseline: 1.0012x; 1.0012x over previous
import jax
import jax.numpy as jnp
from jax import lax
from jax.experimental import pallas as pl
from jax.experimental.pallas import tpu as pltpu
from jax.experimental.pallas import tpu_sc as plsc

VOCAB = 1000000
EMBED_DIM = 64
BATCH = 16384
SEQ = 50

B = BATCH * SEQ
D = EMBED_DIM
NC, NS = 2, 16
NW = NC * NS
IDX_W = 128
CHUNK = 512
ROWS_PER_W = B // NW
STEPS = ROWS_PER_W // CHUNK
IDX_ROWS_PER_STEP = CHUNK // IDX_W
IDX_ROWS_PER_W = ROWS_PER_W // IDX_W


def _gather_body(idx_hbm, table_hbm, out_hbm, idx_all, rows_v,
                 sem_g0, sem_g1, sem_o0, sem_o1):
    wid = lax.axis_index("s") * NC + lax.axis_index("c")
    row_base = wid * ROWS_PER_W
    sem_g = (sem_g0, sem_g1)
    sem_o = (sem_o0, sem_o1)

    idx0 = pl.multiple_of(wid * ROWS_PER_W, 8)
    head = 2 * CHUNK
    pltpu.sync_copy(idx_hbm.at[pl.ds(idx0, head)],
                    idx_all.at[pl.ds(0, head)])

    def fire_gathers(g, buf):
        pltpu.async_copy(
            table_hbm.at[idx_all.at[pl.ds(g * CHUNK, CHUNK)]],
            rows_v.at[buf],
            sem_g[buf],
        )

    def wait_gathers(buf):
        pltpu.make_async_copy(
            table_hbm.at[pl.ds(0, CHUNK)], rows_v.at[buf], sem_g[buf]
        ).wait()

    def fire_writeback(g, buf):
        out0 = pl.multiple_of(row_base + g * CHUNK, CHUNK)
        pltpu.async_copy(rows_v.at[buf], out_hbm.at[pl.ds(out0, CHUNK)],
                         sem_o[buf])

    def drain_writeback(buf):
        pltpu.make_async_copy(
            rows_v.at[buf], out_hbm.at[pl.ds(0, CHUNK)], sem_o[buf]
        ).wait()

    fire_gathers(0, 0)
    pltpu.sync_copy(
        idx_hbm.at[pl.ds(pl.multiple_of(idx0 + head, 8), ROWS_PER_W - head)],
        idx_all.at[pl.ds(head, ROWS_PER_W - head)])

    def pair(i, carry):
        g0 = 2 * i
        g1 = 2 * i + 1

        @pl.when(i >= 1)
        def _():
            drain_writeback(1)

        fire_gathers(g1, 1)
        wait_gathers(0)
        fire_writeback(g0, 0)

        @pl.when(g1 + 1 < STEPS)
        def _():
            drain_writeback(0)
            fire_gathers(g1 + 1, 0)

        wait_gathers(1)
        fire_writeback(g1, 1)
        return carry

    lax.fori_loop(0, STEPS // 2, pair, 0)
    drain_writeback(0)
    drain_writeback(1)


@jax.jit
def _run(idx2d, table):
    mesh = plsc.VectorSubcoreMesh(core_axis_name="c", subcore_axis_name="s")
    k = pl.kernel(
        _gather_body,
        out_type=jax.ShapeDtypeStruct((B, D), jnp.float32),
        mesh=mesh,
        scratch_types=[
            pltpu.VMEM((ROWS_PER_W,), jnp.int32),
            pltpu.VMEM((2, CHUNK, D), jnp.float32),
            pltpu.SemaphoreType.DMA,
            pltpu.SemaphoreType.DMA,
            pltpu.SemaphoreType.DMA,
            pltpu.SemaphoreType.DMA,
        ],
        compiler_params=pltpu.CompilerParams(use_tc_tiling_on_sc=False),
    )
    return k(idx2d, table)


def kernel(chord_pitches, pitch_table):
    idx1d = chord_pitches.reshape(B)
    out = _run(idx1d, pitch_table)
    return out.reshape(BATCH, SEQ, D)
